# baseline (speedup 1.0000x reference)
import functools

import jax
import jax.numpy as jnp
from jax import lax
from jax.experimental import pallas as pl
from jax.experimental.pallas import tpu as pltpu
from jax.experimental.pallas import tpu_sc as plsc

B = 16384
F = 26
V = 100
D = 64

NC = 2
NS = 16
NW = NC * NS

BT = B // 128
BTW = BT // NW
BW = 128 * BTW
NG = BW // 16

_mesh = plsc.VectorSubcoreMesh(core_axis_name="c", subcore_axis_name="s")


@functools.partial(
    pl.kernel,
    mesh=_mesh,
    out_type=jax.ShapeDtypeStruct((F, D // 8, BT, 8, 128), jnp.float32),
    compiler_params=pltpu.CompilerParams(
        use_tc_tiling_on_sc=False, needs_layout_passes=False
    ),
    scratch_types=[
        pltpu.VMEM((F, BW), jnp.int32),
        pltpu.VMEM((2, D, 128), jnp.float32),
        pltpu.VMEM((2, D // 8, BTW, 8, 128), jnp.float32),
        pltpu.VMEM_SHARED((F, D, 128), jnp.float32),
        pltpu.SemaphoreType.DMA,
        pltpu.SemaphoreType.DMA,
    ],
)
def _emb_lookup(idx_hbm, tab_hbm, out_hbm, idx_v, tf, ob, tab_s, tsem, osem):
    cid = lax.axis_index("c")
    sid = lax.axis_index("s")
    wid = sid * NC + cid

    @pl.when(sid == 0)
    def _():
        pltpu.sync_copy(tab_hbm, tab_s)

    pltpu.sync_copy(idx_hbm.at[:, pl.ds(wid * BW, BW)], idx_v)
    plsc.subcore_barrier()

    def prefetch(f, q):
        pltpu.async_copy(tab_s.at[f], tf.at[q], tsem)

    def wait_table(q):
        pltpu.make_async_copy(tab_s.at[0], tf.at[q], tsem).wait()

    def write_out(f, q):
        pltpu.async_copy(
            ob.at[q], out_hbm.at[f, :, pl.ds(wid * BTW, BTW)], osem
        )

    def wait_write(q):
        pltpu.make_async_copy(
            ob.at[q], out_hbm.at[0, :, pl.ds(wid * BTW, BTW)], osem
        ).wait()

    def compute(f, q):
        @plsc.parallel_loop(0, NG, unroll=2)
        def gbody(g):
            btl = g // 8
            jg = (g % 8) * 16
            idxv = idx_v[f, pl.ds(g * 16, 16)]
            for d in range(D):
                val = plsc.load_gather(tf.at[q, d], [idxv])
                ob[q, d // 8, btl, d % 8, pl.ds(jg, 16)] = val

    prefetch(0, 0)

    def step(h, carry):
        for q in range(2):
            f = h * 2 + q
            wait_table(q)

            @pl.when(f + 1 < F)
            def _():
                prefetch(f + 1, 1 - q)

            @pl.when(f >= 2)
            def _():
                wait_write(q)

            compute(f, q)
            write_out(f, q)
        return carry

    lax.fori_loop(0, F // 2, step, 0)

    for q in range(2):
        wait_write(q)


def kernel(inputs, tables):
    idx_t = inputs.T
    tab_t = jnp.pad(tables.transpose(0, 2, 1), ((0, 0), (0, 0), (0, 128 - V)))
    x = _emb_lookup(idx_t, tab_t)
    return x.transpose(2, 4, 0, 1, 3).reshape(B, F, D)


# speedup vs baseline: 1.1145x; 1.1145x over previous
import functools

import jax
import jax.numpy as jnp
from jax import lax
from jax.experimental import pallas as pl
from jax.experimental.pallas import tpu as pltpu
from jax.experimental.pallas import tpu_sc as plsc

B = 16384
F = 26
V = 100
D = 64

NC = 2
NS = 16
NW = NC * NS

BT = B // 128
BTW = BT // NW
BW = 128 * BTW
NG = BW // 16

_mesh = plsc.VectorSubcoreMesh(core_axis_name="c", subcore_axis_name="s")


@functools.partial(
    pl.kernel,
    mesh=_mesh,
    out_type=jax.ShapeDtypeStruct((F, D // 8, BT, 8, 128), jnp.float32),
    compiler_params=pltpu.CompilerParams(
        use_tc_tiling_on_sc=False, needs_layout_passes=False
    ),
    scratch_types=[
        pltpu.VMEM((F, BW), jnp.int32),
        pltpu.VMEM((2, D, 128), jnp.float32),
        pltpu.VMEM((2, D // 8, BTW, 8, 128), jnp.float32),
        pltpu.VMEM_SHARED((F, D, 128), jnp.float32),
        pltpu.SemaphoreType.DMA,
        pltpu.SemaphoreType.DMA,
    ],
)
def _emb_lookup(idx_hbm, tab_hbm, out_hbm, idx_v, tf, ob, tab_s, tsem, osem):
    cid = lax.axis_index("c")
    sid = lax.axis_index("s")
    wid = sid * NC + cid

    @pl.when(sid == 0)
    def _():
        pltpu.sync_copy(tab_hbm, tab_s)

    pltpu.sync_copy(idx_hbm.at[:, pl.ds(wid * BW, BW)], idx_v)
    plsc.subcore_barrier()

    def prefetch(f, q):
        pltpu.async_copy(tab_s.at[f], tf.at[q], tsem)

    def wait_table(q):
        pltpu.make_async_copy(tab_s.at[0], tf.at[q], tsem).wait()

    def write_out(f, q):
        pltpu.async_copy(
            ob.at[q], out_hbm.at[f, :, pl.ds(wid * BTW, BTW)], osem
        )

    def wait_write(q):
        pltpu.make_async_copy(
            ob.at[q], out_hbm.at[0, :, pl.ds(wid * BTW, BTW)], osem
        ).wait()

    def compute(f, q):
        @plsc.parallel_loop(0, NG)
        def gbody(g):
            btl = g // 8
            jg = (g % 8) * 16
            idxv = idx_v[f, pl.ds(g * 16, 16)]
            for d in range(D):
                val = plsc.load_gather(tf.at[q, d], [idxv])
                ob[q, d // 8, btl, d % 8, pl.ds(jg, 16)] = val

    prefetch(0, 0)

    def step(h, carry):
        for q in range(2):
            f = h * 2 + q
            wait_table(q)

            @pl.when(f + 1 < F)
            def _():
                prefetch(f + 1, 1 - q)

            @pl.when(f >= 2)
            def _():
                wait_write(q)

            compute(f, q)
            write_out(f, q)
        return carry

    lax.fori_loop(0, F // 2, step, 0)

    for q in range(2):
        wait_write(q)


def kernel(inputs, tables):
    idx_t = inputs.T
    tab_t = jnp.pad(tables.transpose(0, 2, 1), ((0, 0), (0, 0), (0, 128 - V)))
    x = _emb_lookup(idx_t, tab_t)
    return x.transpose(2, 4, 0, 1, 3).reshape(B, F, D)


# speedup vs baseline: 1.1483x; 1.0304x over previous
import functools

import jax
import jax.numpy as jnp
from jax import lax
from jax.experimental import pallas as pl
from jax.experimental.pallas import tpu as pltpu
from jax.experimental.pallas import tpu_sc as plsc

B = 16384
F = 26
V = 100
D = 64

NC = 2
NS = 16
NW = NC * NS

BT = B // 128
BTW = BT // NW
BW = 128 * BTW
NG = BW // 16
HD = D // 16

_mesh = plsc.VectorSubcoreMesh(core_axis_name="c", subcore_axis_name="s")


@functools.partial(
    pl.kernel,
    mesh=_mesh,
    out_type=jax.ShapeDtypeStruct((F, D // 8, BT, 8, 128), jnp.float32),
    compiler_params=pltpu.CompilerParams(
        use_tc_tiling_on_sc=False, needs_layout_passes=False
    ),
    scratch_types=[
        pltpu.VMEM((F, BW), jnp.int32),
        pltpu.VMEM((2, D, 128), jnp.float32),
        pltpu.VMEM((3, HD, BTW, 8, 128), jnp.float32),
        pltpu.VMEM_SHARED((F, D, 128), jnp.float32),
        pltpu.SemaphoreType.DMA,
        pltpu.SemaphoreType.DMA,
    ],
)
def _emb_lookup(idx_hbm, tab_hbm, out_hbm, idx_v, tf, ob, tab_s, tsem, osem):
    cid = lax.axis_index("c")
    sid = lax.axis_index("s")
    wid = sid * NC + cid

    @pl.when(sid == 0)
    def _():
        pltpu.sync_copy(tab_hbm, tab_s)

    pltpu.sync_copy(idx_hbm.at[:, pl.ds(wid * BW, BW)], idx_v)
    plsc.subcore_barrier()

    def prefetch(f, q):
        pltpu.async_copy(tab_s.at[f], tf.at[q], tsem)

    def wait_table(q):
        pltpu.make_async_copy(tab_s.at[0], tf.at[q], tsem).wait()

    def write_half(f, h, r):
        pltpu.async_copy(
            ob.at[r],
            out_hbm.at[f, pl.ds(h * HD, HD), pl.ds(wid * BTW, BTW)],
            osem,
        )

    def wait_half(r):
        pltpu.make_async_copy(
            ob.at[r], out_hbm.at[0, pl.ds(0, HD), pl.ds(wid * BTW, BTW)], osem
        ).wait()

    def compute_half(f, q, r, h):
        @plsc.parallel_loop(0, NG)
        def gbody(g):
            btl = g // 8
            jg = (g % 8) * 16
            idxv = idx_v[f, pl.ds(g * 16, 16)]
            for dl in range(D // 2):
                d = h * (D // 2) + dl
                val = plsc.load_gather(tf.at[q, d], [idxv])
                ob[r, dl // 8, btl, dl % 8, pl.ds(jg, 16)] = val

    prefetch(0, 0)

    def step(s, carry):
        for q in range(2):
            f = s * 2 + q
            wait_table(q)

            @pl.when(f + 1 < F)
            def _():
                prefetch(f + 1, 1 - q)

            for h in range(2):
                hid = f * 2 + h
                r = lax.rem(hid, 3)

                @pl.when(hid >= 3)
                def _():
                    wait_half(r)

                compute_half(f, q, r, h)
                write_half(f, h, r)
        return carry

    lax.fori_loop(0, F // 2, step, 0)

    for _ in range(3):
        wait_half(0)


def kernel(inputs, tables):
    idx_t = inputs.T
    tab_t = jnp.pad(tables.transpose(0, 2, 1), ((0, 0), (0, 0), (0, 128 - V)))
    x = _emb_lookup(idx_t, tab_t)
    return x.transpose(2, 4, 0, 1, 3).reshape(B, F, D)
